# baseline (speedup 1.0000x reference)
import jax
import jax.numpy as jnp
from jax import lax
from jax.experimental import pallas as pl
from jax.experimental.pallas import tpu as pltpu
from jax.experimental.pallas import tpu_sc as plsc

N_NODES = 100000
NUM_SPECIES = 128
EMBED_DIM = 128
CHUNK = 128
NUM_CORES = 2
NUM_SUBCORES = 16
NUM_WORKERS = NUM_CORES * NUM_SUBCORES
NUM_CHUNKS = -(-N_NODES // CHUNK)
TRIPS = -(-NUM_CHUNKS // NUM_WORKERS)
LAST_START = N_NODES - CHUNK
NBUF = 3
HBM_UNITS = (5, 11, 17, 23)
CROSS_UNITS = tuple(u for u in range(TRIPS) if u not in HBM_UNITS)
NH = len(HBM_UNITS)
HBM_STORE_AT = (6, 10, 14, 18)


def _gather_body(idx_hbm, w_hbm, out_hbm,
                 idx_v, r0, r1, r2, idx_h, h0, h1, h2, h3, w_sh,
                 sem_i, sem_g, sem_s, sem_hi, sem_hg, sem_hs):
    c = lax.axis_index("c")
    s = lax.axis_index("s")
    wid = s * NUM_CORES + c
    rows = [r0, r1, r2]
    hbuf = [h0, h1, h2, h3]
    @pl.when(s == 0)
    def _stage():
        pltpu.sync_copy(w_hbm, w_sh)
    plsc.subcore_barrier()

    def start_of(u):
        return jnp.minimum((wid + u * NUM_WORKERS) * CHUNK, LAST_START)

    def load_idx(t):
        b = t % NBUF
        return pltpu.async_copy(
            idx_hbm.at[pl.ds(start_of(CROSS_UNITS[t]), CHUNK)],
            idx_v.at[b], sem_i.at[b])

    def gather(b):
        return pltpu.async_copy(w_sh.at[idx_v.at[b]], rows[b], sem_g.at[b])

    def store(t):
        b = t % NBUF
        return pltpu.async_copy(
            rows[b], out_hbm.at[pl.ds(start_of(CROSS_UNITS[t]), CHUNK)],
            sem_s.at[b])

    h_hi = [pltpu.async_copy(
        idx_hbm.at[pl.ds(start_of(u), CHUNK)], idx_h.at[k], sem_hi)
        for k, u in enumerate(HBM_UNITS)]
    for h in h_hi:
        h.wait()
    h_hg = [pltpu.async_copy(w_hbm.at[idx_h.at[k]], hbuf[k], sem_hg.at[k])
            for k in range(NH)]

    def store_h(k):
        return pltpu.async_copy(
            hbuf[k], out_hbm.at[pl.ds(start_of(HBM_UNITS[k]), CHUNK)],
            sem_hs.at[k])

    NC = len(CROSS_UNITS)
    h_idx = [None] * NC
    h_s = [None] * NC
    h_hs = [None] * NH

    for t in range(min(NBUF, NC)):
        h_idx[t] = load_idx(t)
    for t in range(NC):
        h_idx[t].wait()
        if t >= NBUF:
            h_s[t - NBUF].wait()
        g = gather(t % NBUF)
        g.wait()
        if t + NBUF < NC:
            h_idx[t + NBUF] = load_idx(t + NBUF)
        h_s[t] = store(t)
        if t in HBM_STORE_AT:
            k = HBM_STORE_AT.index(t)
            h_hg[k].wait()
            h_hs[k] = store_h(k)
    for t in range(max(0, NC - NBUF), NC):
        h_s[t].wait()
    for k in range(NH):
        h_hs[k].wait()


@jax.jit
def _embed(node_specie, w):
    mesh = plsc.VectorSubcoreMesh(
        core_axis_name="c", subcore_axis_name="s",
        num_cores=NUM_CORES, num_subcores=NUM_SUBCORES)
    return pl.kernel(
        _gather_body,
        out_type=jax.ShapeDtypeStruct((N_NODES, EMBED_DIM), jnp.float32),
        mesh=mesh,
        scratch_types=[
            pltpu.VMEM((NBUF, CHUNK), jnp.int32),
            pltpu.VMEM((CHUNK, EMBED_DIM), jnp.float32),
            pltpu.VMEM((CHUNK, EMBED_DIM), jnp.float32),
            pltpu.VMEM((CHUNK, EMBED_DIM), jnp.float32),
            pltpu.VMEM((NH, CHUNK), jnp.int32),
            pltpu.VMEM((CHUNK, EMBED_DIM), jnp.float32),
            pltpu.VMEM((CHUNK, EMBED_DIM), jnp.float32),
            pltpu.VMEM((CHUNK, EMBED_DIM), jnp.float32),
            pltpu.VMEM((CHUNK, EMBED_DIM), jnp.float32),
            pltpu.VMEM_SHARED((NUM_SPECIES, EMBED_DIM), jnp.float32),
            pltpu.SemaphoreType.DMA((NBUF,)),
            pltpu.SemaphoreType.DMA((NBUF,)),
            pltpu.SemaphoreType.DMA((NBUF,)),
            pltpu.SemaphoreType.DMA,
            pltpu.SemaphoreType.DMA((NH,)),
            pltpu.SemaphoreType.DMA((NH,)),
        ],
    )(node_specie, w)


def kernel(node_specie, w):
    return _embed(node_specie.astype(jnp.int32), w)


# speedup vs baseline: 1.3845x; 1.3845x over previous
import jax
import jax.numpy as jnp
from jax import lax
from jax.experimental import pallas as pl
from jax.experimental.pallas import tpu as pltpu
from jax.experimental.pallas import tpu_sc as plsc

N_NODES = 100000
NUM_SPECIES = 128
EMBED_DIM = 128
SUB = 128
SUBS = 1
CHUNK = SUB * SUBS
NUM_CORES = 2
NUM_SUBCORES = 16
NUM_WORKERS = NUM_CORES * NUM_SUBCORES
NUM_CHUNKS = -(-N_NODES // CHUNK)
TRIPS = -(-NUM_CHUNKS // NUM_WORKERS)
LAST_START = N_NODES - CHUNK
NBUF = 3


def _gather_body(idx_hbm, w_hbm, out_hbm,
                 idx_v, rows_v, w_sh, sem_i, sem_g, sem_s):
    c = lax.axis_index("c")
    s = lax.axis_index("s")
    wid = s * NUM_CORES + c
    @pl.when(s == 0)
    def _stage():
        pltpu.sync_copy(w_hbm, w_sh)
    plsc.subcore_barrier()

    def start_of(j):
        return jnp.minimum((wid + j * NUM_WORKERS) * CHUNK, LAST_START)

    def load_idx(j):
        b = j % NBUF
        return [pltpu.async_copy(
            idx_hbm.at[pl.ds(start_of(j) + h * SUB, SUB)],
            idx_v.at[b, h], sem_i.at[b]) for h in range(SUBS)]

    def gather(j, h):
        b = j % NBUF
        return pltpu.async_copy(
            w_sh.at[idx_v.at[b, h]],
            rows_v.at[b, pl.ds(h * SUB, SUB)],
            sem_g.at[b])

    def store(j):
        b = j % NBUF
        return pltpu.async_copy(
            rows_v.at[b], out_hbm.at[pl.ds(start_of(j), CHUNK)], sem_s.at[b])

    h_idx = [None] * TRIPS
    h_s = [None] * TRIPS

    for j in range(min(NBUF, TRIPS)):
        h_idx[j] = load_idx(j)
    for j in range(TRIPS):
        for h in h_idx[j]:
            h.wait()
        if j >= NBUF:
            h_s[j - NBUF].wait()
        hg = [gather(j, h) for h in range(SUBS)]
        for g in hg:
            g.wait()
        if j + NBUF < TRIPS:
            h_idx[j + NBUF] = load_idx(j + NBUF)
        h_s[j] = store(j)
    for j in range(max(0, TRIPS - NBUF), TRIPS):
        h_s[j].wait()


@jax.jit
def _embed(node_specie, w):
    mesh = plsc.VectorSubcoreMesh(
        core_axis_name="c", subcore_axis_name="s",
        num_cores=NUM_CORES, num_subcores=NUM_SUBCORES)
    return pl.kernel(
        _gather_body,
        out_type=jax.ShapeDtypeStruct((N_NODES, EMBED_DIM), jnp.float32),
        mesh=mesh,
        scratch_types=[
            pltpu.VMEM((NBUF, SUBS, SUB), jnp.int32),
            pltpu.VMEM((NBUF, CHUNK, EMBED_DIM), jnp.float32),
            pltpu.VMEM_SHARED((NUM_SPECIES, EMBED_DIM), jnp.float32),
            pltpu.SemaphoreType.DMA((NBUF,)),
            pltpu.SemaphoreType.DMA((NBUF,)),
            pltpu.SemaphoreType.DMA((NBUF,)),
        ],
    )(node_specie, w)


def kernel(node_specie, w):
    return _embed(node_specie.astype(jnp.int32), w)
